# baseline (speedup 1.0000x reference)
import functools

import jax
import jax.numpy as jnp
from jax import lax
from jax.experimental import pallas as pl
from jax.experimental.pallas import tpu as pltpu
from jax.experimental.pallas import tpu_sc as plsc

NC = 2
NS = 16
NW = NC * NS
L = 16


def _gather_grid(b_per_w, n_chunks, chunk, D):
    mesh = plsc.VectorSubcoreMesh(core_axis_name="c", subcore_axis_name="s")
    n_pairs = (n_chunks + 1) // 2
    odd = n_chunks % 2 == 1

    @functools.partial(
        pl.kernel,
        mesh=mesh,
        out_type=jax.ShapeDtypeStruct((NW * b_per_w, D), jnp.float32),
        scratch_types=[
            pltpu.VMEM((2 * chunk,), jnp.int32),
            pltpu.VMEM((2 * chunk,), jnp.int32),
            pltpu.VMEM((chunk,), jnp.int32),
            pltpu.VMEM((chunk,), jnp.int32),
            pltpu.VMEM((chunk, D), jnp.float32),
            pltpu.VMEM((chunk, D), jnp.float32),
            pltpu.SemaphoreType.DMA,
            pltpu.SemaphoreType.DMA,
            pltpu.SemaphoreType.DMA,
            pltpu.SemaphoreType.DMA,
            pltpu.SemaphoreType.DMA,
        ],
    )
    def k(atom_hbm, conn_hbm, out_hbm,
          c0, c1, i0, i1, r0, r1, cs0, cs1, gsem, os0, os1):
        conn_v = (c0, c1)
        idx_v = (i0, i1)
        rows_v = (r0, r1)
        csem = (cs0, cs1)
        osem = (os0, os1)

        wid = lax.axis_index("s") * NC + lax.axis_index("c")
        base_w = wid * b_per_w

        lane = lax.iota(jnp.int32, L)
        odd_perm = (2 * lane + 1) % L
        low_half = lane < (L // 2)
        dnums = lax.GatherDimensionNumbers(
            offset_dims=(), collapsed_slice_dims=(0,), start_index_map=(0,))

        def take16(v):
            return lax.gather(
                v, odd_perm[:, None], dnums, (1,),
                mode=lax.GatherScatterMode.PROMISE_IN_BOUNDS)

        def conn_slice(j):
            return conn_hbm.at[pl.ds(2 * (base_w + j * chunk), 2 * chunk)]

        def out_slice(j):
            return out_hbm.at[pl.ds(base_w + j * chunk, chunk), :]

        def conn_start(j, b):
            pltpu.async_copy(conn_slice(j), conn_v[b], csem[b])

        def conn_wait(j, b):
            pltpu.make_async_copy(conn_slice(j), conn_v[b], csem[b]).wait()

        def out_start(j, b):
            pltpu.async_copy(rows_v[b], out_slice(j), osem[b])

        def out_wait(j, b):
            pltpu.make_async_copy(rows_v[b], out_slice(j), osem[b]).wait()

        conn_start(0, 0)
        conn_start(1, 1)

        def pair(jj, carry):
            for b in (0, 1):
                j = 2 * jj + b

                def sub(b=b, j=j):
                    conn_wait(j, b)
                    for t in range(chunk // L):
                        v0 = conn_v[b][pl.ds(2 * t * L, L)]
                        v1 = conn_v[b][pl.ds((2 * t + 1) * L, L)]
                        idx_v[b][pl.ds(t * L, L)] = jnp.where(
                            low_half, take16(v0), take16(v1))

                    @pl.when(j + 2 < n_chunks)
                    def _():
                        conn_start(j + 2, b)

                    @pl.when(j >= 2)
                    def _():
                        out_wait(j - 2, b)

                    pltpu.async_copy(
                        atom_hbm.at[idx_v[b]], rows_v[b], gsem).wait()
                    out_start(j, b)

                if odd and b == 1:
                    pl.when(j < n_chunks)(sub)
                else:
                    sub()
            return carry

        lax.fori_loop(0, n_pairs, pair, 0)
        out_wait(n_chunks - 2, (n_chunks - 2) % 2)
        out_wait(n_chunks - 1, (n_chunks - 1) % 2)

    return k


def kernel(atom_matrix, connectivity):
    V, D = atom_matrix.shape
    B = connectivity.shape[0]
    assert B % NW == 0
    b_per_w = B // NW
    chunk = 80
    assert b_per_w % chunk == 0 and chunk % L == 0
    n_chunks = b_per_w // chunk
    conn = connectivity.astype(jnp.int32).reshape(-1)
    return _gather_grid(b_per_w, n_chunks, chunk, D)(atom_matrix, conn)


# speedup vs baseline: 1.1336x; 1.1336x over previous
import functools

import jax
import jax.numpy as jnp
from jax import lax
from jax.experimental import pallas as pl
from jax.experimental.pallas import tpu as pltpu
from jax.experimental.pallas import tpu_sc as plsc

NC = 2
NS = 16
NW = NC * NS
L = 16


def _gather_grid(b_per_w, n_chunks, chunk, D):
    mesh = plsc.VectorSubcoreMesh(core_axis_name="c", subcore_axis_name="s")
    n_pairs = (n_chunks + 1) // 2
    odd = n_chunks % 2 == 1

    @functools.partial(
        pl.kernel,
        mesh=mesh,
        out_type=jax.ShapeDtypeStruct((NW * b_per_w, D), jnp.float32),
        scratch_types=[
            pltpu.VMEM((2 * chunk,), jnp.int32),
            pltpu.VMEM((2 * chunk,), jnp.int32),
            pltpu.VMEM((chunk,), jnp.int32),
            pltpu.VMEM((chunk,), jnp.int32),
            pltpu.VMEM((chunk, D), jnp.float32),
            pltpu.VMEM((chunk, D), jnp.float32),
            pltpu.SemaphoreType.DMA,
            pltpu.SemaphoreType.DMA,
            pltpu.SemaphoreType.DMA,
            pltpu.SemaphoreType.DMA,
            pltpu.SemaphoreType.DMA,
            pltpu.SemaphoreType.DMA,
        ],
    )
    def k(atom_hbm, conn_hbm, out_hbm,
          c0, c1, i0, i1, r0, r1, cs0, cs1, gs0, gs1, os0, os1):
        conn_v = (c0, c1)
        idx_v = (i0, i1)
        rows_v = (r0, r1)
        csem = (cs0, cs1)
        gsem = (gs0, gs1)
        osem = (os0, os1)

        wid = lax.axis_index("s") * NC + lax.axis_index("c")
        base_w = wid * b_per_w

        lane = lax.iota(jnp.int32, L)
        odd_perm = (2 * lane + 1) % L
        low_half = lane < (L // 2)
        dnums = lax.GatherDimensionNumbers(
            offset_dims=(), collapsed_slice_dims=(0,), start_index_map=(0,))

        def take16(v):
            return lax.gather(
                v, odd_perm[:, None], dnums, (1,),
                mode=lax.GatherScatterMode.PROMISE_IN_BOUNDS)

        def conn_slice(j):
            return conn_hbm.at[pl.ds(2 * (base_w + j * chunk), 2 * chunk)]

        def out_slice(j):
            return out_hbm.at[pl.ds(base_w + j * chunk, chunk), :]

        def conn_start(j, b):
            pltpu.async_copy(conn_slice(j), conn_v[b], csem[b])

        def conn_wait(j, b):
            pltpu.make_async_copy(conn_slice(j), conn_v[b], csem[b]).wait()

        def out_start(j, b):
            pltpu.async_copy(rows_v[b], out_slice(j), osem[b])

        def out_wait(j, b):
            pltpu.make_async_copy(rows_v[b], out_slice(j), osem[b]).wait()

        def gather_start(b):
            pltpu.async_copy(atom_hbm.at[idx_v[b]], rows_v[b], gsem[b])

        def gather_wait(b):
            pltpu.make_async_copy(
                atom_hbm.at[idx_v[b]], rows_v[b], gsem[b]).wait()

        conn_start(0, 0)
        conn_start(1, 1)

        def pair(jj, carry):
            for b in (0, 1):
                j = 2 * jj + b

                def sub(b=b, j=j):
                    conn_wait(j, b)
                    for t in range(chunk // L):
                        v0 = conn_v[b][pl.ds(2 * t * L, L)]
                        v1 = conn_v[b][pl.ds((2 * t + 1) * L, L)]
                        idx_v[b][pl.ds(t * L, L)] = jnp.where(
                            low_half, take16(v0), take16(v1))

                    @pl.when(j + 2 < n_chunks)
                    def _():
                        conn_start(j + 2, b)

                    @pl.when(j >= 2)
                    def _():
                        out_wait(j - 2, b)

                    gather_start(b)

                    @pl.when(j >= 1)
                    def _():
                        gather_wait(1 - b)
                        out_start(j - 1, 1 - b)

                if odd and b == 1:
                    pl.when(j < n_chunks)(sub)
                else:
                    sub()
            return carry

        lax.fori_loop(0, n_pairs, pair, 0)
        jl = n_chunks - 1
        gather_wait(jl % 2)
        out_start(jl, jl % 2)
        out_wait(jl - 1, (jl - 1) % 2)
        out_wait(jl, jl % 2)

    return k


def kernel(atom_matrix, connectivity):
    V, D = atom_matrix.shape
    B = connectivity.shape[0]
    assert B % NW == 0
    b_per_w = B // NW
    chunk = 80
    assert b_per_w % chunk == 0 and chunk % L == 0
    n_chunks = b_per_w // chunk
    conn = connectivity.astype(jnp.int32).reshape(-1)
    return _gather_grid(b_per_w, n_chunks, chunk, D)(atom_matrix, conn)


# speedup vs baseline: 1.1780x; 1.0391x over previous
import functools

import jax
import jax.numpy as jnp
from jax import lax
from jax.experimental import pallas as pl
from jax.experimental.pallas import tpu as pltpu
from jax.experimental.pallas import tpu_sc as plsc

NC = 2
NS = 16
NW = NC * NS
L = 16


def _gather_grid(b_per_w, n_chunks, chunk, D):
    mesh = plsc.VectorSubcoreMesh(core_axis_name="c", subcore_axis_name="s")
    n_pairs = (n_chunks + 1) // 2
    odd = n_chunks % 2 == 1

    @functools.partial(
        pl.kernel,
        mesh=mesh,
        out_type=jax.ShapeDtypeStruct((NW * b_per_w, D), jnp.float32),
        scratch_types=[
            pltpu.VMEM((2 * chunk,), jnp.int32),
            pltpu.VMEM((2 * chunk,), jnp.int32),
            pltpu.VMEM((chunk,), jnp.int32),
            pltpu.VMEM((chunk,), jnp.int32),
            pltpu.VMEM((chunk, D), jnp.float32),
            pltpu.VMEM((chunk, D), jnp.float32),
            pltpu.SemaphoreType.DMA,
            pltpu.SemaphoreType.DMA,
            pltpu.SemaphoreType.DMA,
            pltpu.SemaphoreType.DMA,
            pltpu.SemaphoreType.DMA,
            pltpu.SemaphoreType.DMA,
        ],
    )
    def k(atom_hbm, conn_hbm, out_hbm,
          c0, c1, i0, i1, r0, r1, cs0, cs1, gs0, gs1, os0, os1):
        conn_v = (c0, c1)
        idx_v = (i0, i1)
        rows_v = (r0, r1)
        csem = (cs0, cs1)
        gsem = (gs0, gs1)
        osem = (os0, os1)

        wid = lax.axis_index("s") * NC + lax.axis_index("c")
        base_w = wid * b_per_w

        lane = lax.iota(jnp.int32, L)
        odd_perm = (2 * lane + 1) % L
        low_half = lane < (L // 2)
        dnums = lax.GatherDimensionNumbers(
            offset_dims=(), collapsed_slice_dims=(0,), start_index_map=(0,))

        def take16(v):
            return lax.gather(
                v, odd_perm[:, None], dnums, (1,),
                mode=lax.GatherScatterMode.PROMISE_IN_BOUNDS)

        def conn_slice(j):
            return conn_hbm.at[pl.ds(2 * (base_w + j * chunk), 2 * chunk)]

        def out_slice(j):
            return out_hbm.at[pl.ds(base_w + j * chunk, chunk), :]

        def conn_start(j, b):
            pltpu.async_copy(conn_slice(j), conn_v[b], csem[b])

        def conn_wait(j, b):
            pltpu.make_async_copy(conn_slice(j), conn_v[b], csem[b]).wait()

        def out_start(j, b):
            pltpu.async_copy(rows_v[b], out_slice(j), osem[b])

        def out_wait(j, b):
            pltpu.make_async_copy(rows_v[b], out_slice(j), osem[b]).wait()

        def gather_start(b):
            pltpu.async_copy(atom_hbm.at[idx_v[b]], rows_v[b], gsem[b])

        def gather_wait(b):
            pltpu.make_async_copy(
                atom_hbm.at[idx_v[b]], rows_v[b], gsem[b]).wait()

        conn_start(0, 0)
        conn_start(1, 1)

        def pair(jj, carry):
            for b in (0, 1):
                j = 2 * jj + b

                def sub(b=b, j=j):
                    conn_wait(j, b)
                    for t in range(chunk // L):
                        v0 = conn_v[b][pl.ds(2 * t * L, L)]
                        v1 = conn_v[b][pl.ds((2 * t + 1) * L, L)]
                        idx_v[b][pl.ds(t * L, L)] = jnp.where(
                            low_half, take16(v0), take16(v1))

                    @pl.when(j + 2 < n_chunks)
                    def _():
                        conn_start(j + 2, b)

                    @pl.when(j >= 2)
                    def _():
                        out_wait(j - 2, b)

                    gather_start(b)

                    @pl.when(j >= 1)
                    def _():
                        gather_wait(1 - b)
                        out_start(j - 1, 1 - b)

                if odd and b == 1:
                    pl.when(j < n_chunks)(sub)
                else:
                    sub()
            return carry

        lax.fori_loop(0, n_pairs, pair, 0)
        jl = n_chunks - 1
        gather_wait(jl % 2)
        out_start(jl, jl % 2)
        out_wait(jl - 1, (jl - 1) % 2)
        out_wait(jl, jl % 2)

    return k


def kernel(atom_matrix, connectivity):
    V, D = atom_matrix.shape
    B = connectivity.shape[0]
    assert B % NW == 0
    b_per_w = B // NW
    chunk = 400
    assert b_per_w % chunk == 0 and chunk % L == 0
    n_chunks = b_per_w // chunk
    conn = connectivity.astype(jnp.int32).reshape(-1)
    return _gather_grid(b_per_w, n_chunks, chunk, D)(atom_matrix, conn)


# speedup vs baseline: 2.4898x; 2.1136x over previous
import functools

import jax
import jax.numpy as jnp
from jax import lax
from jax.experimental import pallas as pl
from jax.experimental.pallas import tpu as pltpu
from jax.experimental.pallas import tpu_sc as plsc

NC = 2
NS = 16
NW = NC * NS
L = 16


def _gather_grid(b_per_w, n_chunks, chunk, D):
    mesh = plsc.VectorSubcoreMesh(core_axis_name="c", subcore_axis_name="s")

    @functools.partial(
        pl.kernel,
        mesh=mesh,
        out_type=jax.ShapeDtypeStruct((NW * b_per_w, D), jnp.float32),
        scratch_types=[
            pltpu.VMEM((chunk,), jnp.int32),
            pltpu.VMEM((chunk,), jnp.int32),
            pltpu.VMEM((chunk,), jnp.int32),
            pltpu.VMEM((chunk, D), jnp.float32),
            pltpu.VMEM((chunk, D), jnp.float32),
            pltpu.SemaphoreType.DMA,
            pltpu.SemaphoreType.DMA,
            pltpu.SemaphoreType.DMA,
            pltpu.SemaphoreType.DMA,
            pltpu.SemaphoreType.DMA,
            pltpu.SemaphoreType.DMA,
            pltpu.SemaphoreType.DMA,
        ],
    )
    def k(atom_hbm, idx_hbm, out_hbm,
          i0, i1, i2, r0, r1, cs0, cs1, cs2, gs0, gs1, os0, os1):
        idx_v = (i0, i1, i2)
        rows_v = (r0, r1)
        csem = (cs0, cs1, cs2)
        gsem = (gs0, gs1)
        osem = (os0, os1)

        wid = lax.axis_index("s") * NC + lax.axis_index("c")
        base_w = wid * b_per_w

        def idx_slice(j):
            return idx_hbm.at[pl.ds(base_w + j * chunk, chunk)]

        def out_slice(j):
            return out_hbm.at[pl.ds(base_w + j * chunk, chunk), :]

        def conn_start(j):
            pltpu.async_copy(idx_slice(j), idx_v[j % 3], csem[j % 3])

        def conn_wait(j):
            pltpu.make_async_copy(idx_slice(j), idx_v[j % 3], csem[j % 3]).wait()

        def out_start(j):
            pltpu.async_copy(rows_v[j % 2], out_slice(j), osem[j % 2])

        def out_wait(j):
            pltpu.make_async_copy(rows_v[j % 2], out_slice(j), osem[j % 2]).wait()

        def gather_start(j):
            pltpu.async_copy(
                atom_hbm.at[idx_v[j % 3]], rows_v[j % 2], gsem[j % 2])

        def gather_wait(j):
            pltpu.make_async_copy(
                atom_hbm.at[idx_v[j % 3]], rows_v[j % 2], gsem[j % 2]).wait()

        for j in range(min(3, n_chunks)):
            conn_start(j)

        for j in range(n_chunks):
            conn_wait(j)
            if j >= 2:
                out_wait(j - 2)
            gather_start(j)
            if j >= 1:
                gather_wait(j - 1)
                out_start(j - 1)
                if j + 2 < n_chunks:
                    conn_start(j + 2)

        gather_wait(n_chunks - 1)
        out_start(n_chunks - 1)
        if n_chunks >= 2:
            out_wait(n_chunks - 2)
        out_wait(n_chunks - 1)

    return k


def kernel(atom_matrix, connectivity):
    V, D = atom_matrix.shape
    B = connectivity.shape[0]
    assert B % NW == 0
    b_per_w = B // NW
    chunk = 400
    assert b_per_w % chunk == 0 and chunk % L == 0
    n_chunks = b_per_w // chunk
    idx = connectivity[:, 1].astype(jnp.int32)
    return _gather_grid(b_per_w, n_chunks, chunk, D)(atom_matrix, idx)
